# initial kernel scaffold (unmeasured)
import jax
import jax.numpy as jnp
from jax import lax
from jax.experimental import pallas as pl
from jax.experimental.pallas import tpu as pltpu

N_DEV = 8
B = 2
S_LOC = 128
S_GLB = 1024
D = 512
HQ = 4
DH = 64
HD = HQ * DH


def kernel(x, Wq, Wk, Wv, Wo):
    def body(x_ref, wq_ref, wk_ref, wv_ref, wo_ref, out_ref,
             xg_ref, part_ref, sbuf_ref, rsc_ref,
             ag_send, ag_recv, rs_send, rs_recv):
        me = lax.axis_index("i")
        left = (me - 1) % N_DEV
        right = (me + 1) % N_DEV

        xg_ref[me] = x_ref[...]

        barrier = pltpu.get_barrier_semaphore()
        for nbr in (left, right):
            pl.semaphore_signal(barrier, inc=1, device_id=(nbr,),
                                device_id_type=pl.DeviceIdType.MESH)
        pl.semaphore_wait(barrier, 2)

        for h in range(N_DEV - 1):
            c_s = (me - h) % N_DEV
            rdma = pltpu.make_async_remote_copy(
                src_ref=xg_ref.at[c_s],
                dst_ref=xg_ref.at[c_s],
                send_sem=ag_send.at[h],
                recv_sem=ag_recv.at[h],
                device_id=(right,),
                device_id_type=pl.DeviceIdType.MESH,
            )
            rdma.start()
            rdma.wait()

        lane = lax.broadcasted_iota(jnp.int32, (S_GLB, HD), 1)
        posr = lax.broadcasted_iota(jnp.float32, (S_GLB, HD), 0)
        j2 = ((lane % DH) // 2).astype(jnp.float32) * 2.0
        inv = jnp.exp(j2 * (-jnp.log(10000.0) / DH))
        ang = posr * inv
        cos = jnp.cos(ang)
        sins = jnp.sin(ang) * jnp.where(lane % 2 == 0, -1.0, 1.0)
        prow = lax.broadcasted_iota(jnp.int32, (HD, HD), 0)
        pcol = lax.broadcasted_iota(jnp.int32, (HD, HD), 1)
        perm = (prow == (pcol + 1 - 2 * (pcol % 2))).astype(jnp.float32)

        wq = wq_ref[...]
        wk = wk_ref[...]
        wv = wv_ref[...]
        wo = wo_ref[...]
        for b in range(B):
            xb = jnp.concatenate([xg_ref[c, b] for c in range(N_DEV)], axis=0)
            q = jnp.dot(xb, wq, preferred_element_type=jnp.float32)
            k = jnp.dot(xb, wk, preferred_element_type=jnp.float32)
            v = jnp.dot(xb, wv, preferred_element_type=jnp.float32)
            q = q * cos + jnp.dot(q, perm, preferred_element_type=jnp.float32) * sins
            k = k * cos + jnp.dot(k, perm, preferred_element_type=jnp.float32) * sins
            ob = jnp.zeros((S_GLB, D), jnp.float32)
            for hh in range(HQ):
                sl = slice(hh * DH, (hh + 1) * DH)
                s = lax.dot_general(
                    q[:, sl], k[:, sl], (((1,), (1,)), ((), ())),
                    preferred_element_type=jnp.float32,
                ) * 0.125
                s = s - jnp.max(s, axis=-1, keepdims=True)
                w = jnp.exp(s)
                w = w / jnp.sum(w, axis=-1, keepdims=True)
                ctx = jnp.dot(w, v[:, sl], preferred_element_type=jnp.float32)
                ob = ob + jnp.dot(ctx, wo[sl, :],
                                  preferred_element_type=jnp.float32)
            for c in range(N_DEV):
                part_ref[c, b] = ob[c * S_LOC:(c + 1) * S_LOC]

        acc = part_ref[left]
        for t in range(N_DEV - 1):
            sbuf_ref[...] = acc
            rdma = pltpu.make_async_remote_copy(
                src_ref=sbuf_ref,
                dst_ref=rsc_ref.at[t],
                send_sem=rs_send.at[t],
                recv_sem=rs_recv.at[t],
                device_id=(right,),
                device_id_type=pl.DeviceIdType.MESH,
            )
            rdma.start()
            rdma.wait()
            c_r = (me - t - 2) % N_DEV
            acc = rsc_ref[t] + part_ref[c_r]
        out_ref[...] = acc

    return pl.pallas_call(
        body,
        out_shape=jax.ShapeDtypeStruct((B, S_LOC, D), jnp.float32),
        in_specs=[pl.BlockSpec(memory_space=pltpu.VMEM)] * 5,
        out_specs=pl.BlockSpec(memory_space=pltpu.VMEM),
        scratch_shapes=[
            pltpu.VMEM((N_DEV, B, S_LOC, D), jnp.float32),
            pltpu.VMEM((N_DEV, B, S_LOC, D), jnp.float32),
            pltpu.VMEM((B, S_LOC, D), jnp.float32),
            pltpu.VMEM((N_DEV - 1, B, S_LOC, D), jnp.float32),
            pltpu.SemaphoreType.DMA((N_DEV - 1,)),
            pltpu.SemaphoreType.DMA((N_DEV - 1,)),
            pltpu.SemaphoreType.DMA((N_DEV - 1,)),
            pltpu.SemaphoreType.DMA((N_DEV - 1,)),
        ],
        compiler_params=pltpu.CompilerParams(collective_id=0),
    )(x, Wq, Wk, Wv, Wo)


# baseline (device time: 141884 ns/iter reference)
import jax
import jax.numpy as jnp
from jax import lax
from jax.experimental import pallas as pl
from jax.experimental.pallas import tpu as pltpu

N_DEV = 8
B = 2
S_LOC = 128
S_GLB = 1024
D = 512
HQ = 4
DH = 64
HD = HQ * DH


def kernel(x, Wq, Wk, Wv, Wo):
    def body(x_ref, wq_ref, wk_ref, wv_ref, wo_ref, out_ref,
             xg_ref, part_ref, sbuf_ref, rsc_ref,
             ag_send, ag_recv, rs_send, rs_recv):
        me = lax.axis_index("i")
        left = (me - 1) % N_DEV
        right = (me + 1) % N_DEV

        xg_ref[me] = x_ref[...]

        barrier = pltpu.get_barrier_semaphore()
        for nbr in (left, right):
            pl.semaphore_signal(barrier, inc=1, device_id=(nbr,),
                                device_id_type=pl.DeviceIdType.MESH)
        pl.semaphore_wait(barrier, 2)

        for h in range(N_DEV - 1):
            c_s = (me - h) % N_DEV
            rdma = pltpu.make_async_remote_copy(
                src_ref=xg_ref.at[c_s],
                dst_ref=xg_ref.at[c_s],
                send_sem=ag_send.at[h],
                recv_sem=ag_recv.at[h],
                device_id=(right,),
                device_id_type=pl.DeviceIdType.MESH,
            )
            rdma.start()
            rdma.wait()

        lane = lax.broadcasted_iota(jnp.int32, (S_GLB, HD), 1)
        posr = lax.broadcasted_iota(jnp.int32, (S_GLB, HD), 0).astype(jnp.float32)
        j2 = ((lane % DH) // 2).astype(jnp.float32) * 2.0
        inv = jnp.exp(j2 * (-jnp.log(10000.0) / DH))
        ang = posr * inv
        cos = jnp.cos(ang)
        sins = jnp.sin(ang) * jnp.where(lane % 2 == 0, -1.0, 1.0)
        prow = lax.broadcasted_iota(jnp.int32, (HD, HD), 0)
        pcol = lax.broadcasted_iota(jnp.int32, (HD, HD), 1)
        perm = (prow == (pcol + 1 - 2 * (pcol % 2))).astype(jnp.float32)

        wq = wq_ref[...]
        wk = wk_ref[...]
        wv = wv_ref[...]
        wo = wo_ref[...]
        for b in range(B):
            xb = jnp.concatenate([xg_ref[c, b] for c in range(N_DEV)], axis=0)
            q = jnp.dot(xb, wq, preferred_element_type=jnp.float32)
            k = jnp.dot(xb, wk, preferred_element_type=jnp.float32)
            v = jnp.dot(xb, wv, preferred_element_type=jnp.float32)
            q = q * cos + jnp.dot(q, perm, preferred_element_type=jnp.float32) * sins
            k = k * cos + jnp.dot(k, perm, preferred_element_type=jnp.float32) * sins
            ob = jnp.zeros((S_GLB, D), jnp.float32)
            for hh in range(HQ):
                sl = slice(hh * DH, (hh + 1) * DH)
                s = lax.dot_general(
                    q[:, sl], k[:, sl], (((1,), (1,)), ((), ())),
                    preferred_element_type=jnp.float32,
                ) * 0.125
                s = s - jnp.max(s, axis=-1, keepdims=True)
                w = jnp.exp(s)
                w = w / jnp.sum(w, axis=-1, keepdims=True)
                ctx = jnp.dot(w, v[:, sl], preferred_element_type=jnp.float32)
                ob = ob + jnp.dot(ctx, wo[sl, :],
                                  preferred_element_type=jnp.float32)
            for c in range(N_DEV):
                part_ref[c, b] = ob[c * S_LOC:(c + 1) * S_LOC]

        acc = part_ref[left]
        for t in range(N_DEV - 1):
            sbuf_ref[...] = acc
            rdma = pltpu.make_async_remote_copy(
                src_ref=sbuf_ref,
                dst_ref=rsc_ref.at[t],
                send_sem=rs_send.at[t],
                recv_sem=rs_recv.at[t],
                device_id=(right,),
                device_id_type=pl.DeviceIdType.MESH,
            )
            rdma.start()
            rdma.wait()
            c_r = (me - t - 2) % N_DEV
            acc = rsc_ref[t] + part_ref[c_r]
        out_ref[...] = acc

    return pl.pallas_call(
        body,
        out_shape=jax.ShapeDtypeStruct((B, S_LOC, D), jnp.float32),
        in_specs=[pl.BlockSpec(memory_space=pltpu.VMEM)] * 5,
        out_specs=pl.BlockSpec(memory_space=pltpu.VMEM),
        scratch_shapes=[
            pltpu.VMEM((N_DEV, B, S_LOC, D), jnp.float32),
            pltpu.VMEM((N_DEV, B, S_LOC, D), jnp.float32),
            pltpu.VMEM((B, S_LOC, D), jnp.float32),
            pltpu.VMEM((N_DEV - 1, B, S_LOC, D), jnp.float32),
            pltpu.SemaphoreType.DMA((N_DEV - 1,)),
            pltpu.SemaphoreType.DMA((N_DEV - 1,)),
            pltpu.SemaphoreType.DMA((N_DEV - 1,)),
            pltpu.SemaphoreType.DMA((N_DEV - 1,)),
        ],
        compiler_params=pltpu.CompilerParams(collective_id=0),
    )(x, Wq, Wk, Wv, Wo)


# device time: 101040 ns/iter; 1.4042x vs baseline; 1.4042x over previous
import jax
import jax.numpy as jnp
from jax import lax
from jax.experimental import pallas as pl
from jax.experimental.pallas import tpu as pltpu

N_DEV = 8
B = 2
S_LOC = 128
S_GLB = 1024
D = 512
HQ = 4
DH = 64
HD = HQ * DH


def kernel(x, Wq, Wk, Wv, Wo):
    def body(x_ref, wq_ref, wk_ref, wv_ref, wo_ref, out_ref,
             xg_ref, part_ref, sbuf_ref, rsc_ref,
             ag_send, ag_recv, rs_send, rs_recv):
        me = lax.axis_index("i")
        left = (me - 1) % N_DEV
        right = (me + 1) % N_DEV

        xg_ref[me] = x_ref[...]

        barrier = pltpu.get_barrier_semaphore()
        for nbr in (left, right):
            pl.semaphore_signal(barrier, inc=1, device_id=(nbr,),
                                device_id_type=pl.DeviceIdType.MESH)
        pl.semaphore_wait(barrier, 2)

        for h in range(4):
            c_r = (me - h) % N_DEV
            rdma_r = pltpu.make_async_remote_copy(
                src_ref=xg_ref.at[c_r],
                dst_ref=xg_ref.at[c_r],
                send_sem=ag_send.at[h],
                recv_sem=ag_recv.at[h],
                device_id=(right,),
                device_id_type=pl.DeviceIdType.MESH,
            )
            rdma_r.start()
            if h < 3:
                c_l = (me + h) % N_DEV
                rdma_l = pltpu.make_async_remote_copy(
                    src_ref=xg_ref.at[c_l],
                    dst_ref=xg_ref.at[c_l],
                    send_sem=ag_send.at[4 + h],
                    recv_sem=ag_recv.at[4 + h],
                    device_id=(left,),
                    device_id_type=pl.DeviceIdType.MESH,
                )
                rdma_l.start()
                rdma_l.wait()
            rdma_r.wait()

        lane = lax.broadcasted_iota(jnp.int32, (S_GLB, HD), 1)
        posr = lax.broadcasted_iota(jnp.int32, (S_GLB, HD), 0).astype(jnp.float32)
        j2 = ((lane % DH) // 2).astype(jnp.float32) * 2.0
        inv = jnp.exp(j2 * (-jnp.log(10000.0) / DH))
        ang = posr * inv
        cos = jnp.cos(ang)
        sins = jnp.sin(ang) * jnp.where(lane % 2 == 0, -1.0, 1.0)
        prow = lax.broadcasted_iota(jnp.int32, (HD, HD), 0)
        pcol = lax.broadcasted_iota(jnp.int32, (HD, HD), 1)
        perm = (prow == (pcol + 1 - 2 * (pcol % 2))).astype(jnp.float32)

        wq = wq_ref[...]
        wk = wk_ref[...]
        wv = wv_ref[...]
        wo = wo_ref[...]
        for b in range(B):
            xb = jnp.concatenate([xg_ref[c, b] for c in range(N_DEV)], axis=0)
            q = jnp.dot(xb, wq, preferred_element_type=jnp.float32)
            k = jnp.dot(xb, wk, preferred_element_type=jnp.float32)
            v = jnp.dot(xb, wv, preferred_element_type=jnp.float32)
            q = q * cos + jnp.dot(q, perm, preferred_element_type=jnp.float32) * sins
            k = k * cos + jnp.dot(k, perm, preferred_element_type=jnp.float32) * sins
            ob = jnp.zeros((S_GLB, D), jnp.float32)
            for hh in range(HQ):
                sl = slice(hh * DH, (hh + 1) * DH)
                s = lax.dot_general(
                    q[:, sl], k[:, sl], (((1,), (1,)), ((), ())),
                    preferred_element_type=jnp.float32,
                ) * 0.125
                s = s - jnp.max(s, axis=-1, keepdims=True)
                w = jnp.exp(s)
                w = w / jnp.sum(w, axis=-1, keepdims=True)
                ctx = jnp.dot(w, v[:, sl], preferred_element_type=jnp.float32)
                ob = ob + jnp.dot(ctx, wo[sl, :],
                                  preferred_element_type=jnp.float32)
            for c in range(N_DEV):
                part_ref[c, b] = ob[c * S_LOC:(c + 1) * S_LOC]

        for t in range(4):
            if t == 0:
                acc_r = part_ref[(me + 4) % N_DEV]
            else:
                acc_r = rsc_ref[t - 1] + part_ref[(me + 4 - t) % N_DEV]
            sbuf_ref[0] = acc_r
            rdma_r = pltpu.make_async_remote_copy(
                src_ref=sbuf_ref.at[0],
                dst_ref=rsc_ref.at[t],
                send_sem=rs_send.at[t],
                recv_sem=rs_recv.at[t],
                device_id=(right,),
                device_id_type=pl.DeviceIdType.MESH,
            )
            rdma_r.start()
            if t < 3:
                if t == 0:
                    acc_l = part_ref[(me - 3) % N_DEV]
                else:
                    acc_l = rsc_ref[4 + t - 1] + part_ref[(me - 3 + t) % N_DEV]
                sbuf_ref[1] = acc_l
                rdma_l = pltpu.make_async_remote_copy(
                    src_ref=sbuf_ref.at[1],
                    dst_ref=rsc_ref.at[4 + t],
                    send_sem=rs_send.at[4 + t],
                    recv_sem=rs_recv.at[4 + t],
                    device_id=(left,),
                    device_id_type=pl.DeviceIdType.MESH,
                )
                rdma_l.start()
                rdma_l.wait()
            rdma_r.wait()
        out_ref[...] = rsc_ref[3] + rsc_ref[6] + part_ref[me]

    return pl.pallas_call(
        body,
        out_shape=jax.ShapeDtypeStruct((B, S_LOC, D), jnp.float32),
        in_specs=[pl.BlockSpec(memory_space=pltpu.VMEM)] * 5,
        out_specs=pl.BlockSpec(memory_space=pltpu.VMEM),
        scratch_shapes=[
            pltpu.VMEM((N_DEV, B, S_LOC, D), jnp.float32),
            pltpu.VMEM((N_DEV, B, S_LOC, D), jnp.float32),
            pltpu.VMEM((2, B, S_LOC, D), jnp.float32),
            pltpu.VMEM((N_DEV - 1, B, S_LOC, D), jnp.float32),
            pltpu.SemaphoreType.DMA((N_DEV - 1,)),
            pltpu.SemaphoreType.DMA((N_DEV - 1,)),
            pltpu.SemaphoreType.DMA((N_DEV - 1,)),
            pltpu.SemaphoreType.DMA((N_DEV - 1,)),
        ],
        compiler_params=pltpu.CompilerParams(collective_id=0),
    )(x, Wq, Wk, Wv, Wo)


# device time: 78508 ns/iter; 1.8073x vs baseline; 1.2870x over previous
import jax
import jax.numpy as jnp
from jax import lax
from jax.experimental import pallas as pl
from jax.experimental.pallas import tpu as pltpu

N_DEV = 8
B = 2
S_LOC = 128
S_GLB = 1024
D = 512
HQ = 4
DH = 64
HD = HQ * DH


def kernel(x, Wq, Wk, Wv, Wo):
    def body(x_ref, wq_ref, wk_ref, wv_ref, wo_ref, out_ref,
             xg_ref, part_ref, sbuf_ref, rsc_ref,
             ag_send, ag_recv, rs_send, rs_recv):
        me = lax.axis_index("i")
        left = (me - 1) % N_DEV
        right = (me + 1) % N_DEV

        xg_ref[me] = x_ref[...].astype(jnp.bfloat16)

        barrier = pltpu.get_barrier_semaphore()
        for nbr in (left, right):
            pl.semaphore_signal(barrier, inc=1, device_id=(nbr,),
                                device_id_type=pl.DeviceIdType.MESH)
        pl.semaphore_wait(barrier, 2)

        for h in range(4):
            c_r = (me - h) % N_DEV
            rdma_r = pltpu.make_async_remote_copy(
                src_ref=xg_ref.at[c_r],
                dst_ref=xg_ref.at[c_r],
                send_sem=ag_send.at[h],
                recv_sem=ag_recv.at[h],
                device_id=(right,),
                device_id_type=pl.DeviceIdType.MESH,
            )
            rdma_r.start()
            if h < 3:
                c_l = (me + h) % N_DEV
                rdma_l = pltpu.make_async_remote_copy(
                    src_ref=xg_ref.at[c_l],
                    dst_ref=xg_ref.at[c_l],
                    send_sem=ag_send.at[4 + h],
                    recv_sem=ag_recv.at[4 + h],
                    device_id=(left,),
                    device_id_type=pl.DeviceIdType.MESH,
                )
                rdma_l.start()
                rdma_l.wait()
            rdma_r.wait()

        lane = lax.broadcasted_iota(jnp.int32, (S_GLB, HD), 1)
        posr = lax.broadcasted_iota(jnp.int32, (S_GLB, HD), 0).astype(jnp.float32)
        j2 = ((lane % DH) // 2).astype(jnp.float32) * 2.0
        inv = jnp.exp(j2 * (-jnp.log(10000.0) / DH))
        ang = posr * inv
        cos = jnp.cos(ang)
        sins = jnp.sin(ang) * jnp.where(lane % 2 == 0, -1.0, 1.0)
        prow = lax.broadcasted_iota(jnp.int32, (HD, HD), 0)
        pcol = lax.broadcasted_iota(jnp.int32, (HD, HD), 1)
        perm = (prow == (pcol + 1 - 2 * (pcol % 2))).astype(jnp.float32)

        wq = wq_ref[...]
        wk = wk_ref[...]
        wv = wv_ref[...]
        wo = wo_ref[...]
        for b in range(B):
            xb = jnp.concatenate(
                [xg_ref[c, b] for c in range(N_DEV)], axis=0
            ).astype(jnp.float32)
            q = jnp.dot(xb, wq, preferred_element_type=jnp.float32)
            k = jnp.dot(xb, wk, preferred_element_type=jnp.float32)
            v = jnp.dot(xb, wv, preferred_element_type=jnp.float32)
            q = q * cos + jnp.dot(q, perm, preferred_element_type=jnp.float32) * sins
            k = k * cos + jnp.dot(k, perm, preferred_element_type=jnp.float32) * sins
            ob = jnp.zeros((S_GLB, D), jnp.float32)
            for hh in range(HQ):
                sl = slice(hh * DH, (hh + 1) * DH)
                s = lax.dot_general(
                    q[:, sl], k[:, sl], (((1,), (1,)), ((), ())),
                    preferred_element_type=jnp.float32,
                ) * 0.125
                s = s - jnp.max(s, axis=-1, keepdims=True)
                w = jnp.exp(s)
                w = w / jnp.sum(w, axis=-1, keepdims=True)
                ctx = jnp.dot(w, v[:, sl], preferred_element_type=jnp.float32)
                ob = ob + jnp.dot(ctx, wo[sl, :],
                                  preferred_element_type=jnp.float32)
            for c in range(N_DEV):
                part_ref[c, b] = ob[c * S_LOC:(c + 1) * S_LOC].astype(jnp.bfloat16)

        for t in range(4):
            if t == 0:
                acc_r = part_ref[(me + 4) % N_DEV]
            else:
                acc_r = (
                    rsc_ref[t - 1].astype(jnp.float32)
                    + part_ref[(me + 4 - t) % N_DEV].astype(jnp.float32)
                ).astype(jnp.bfloat16)
            sbuf_ref[0] = acc_r
            rdma_r = pltpu.make_async_remote_copy(
                src_ref=sbuf_ref.at[0],
                dst_ref=rsc_ref.at[t],
                send_sem=rs_send.at[t],
                recv_sem=rs_recv.at[t],
                device_id=(right,),
                device_id_type=pl.DeviceIdType.MESH,
            )
            rdma_r.start()
            if t < 3:
                if t == 0:
                    acc_l = part_ref[(me - 3) % N_DEV]
                else:
                    acc_l = (
                        rsc_ref[4 + t - 1].astype(jnp.float32)
                        + part_ref[(me - 3 + t) % N_DEV].astype(jnp.float32)
                    ).astype(jnp.bfloat16)
                sbuf_ref[1] = acc_l
                rdma_l = pltpu.make_async_remote_copy(
                    src_ref=sbuf_ref.at[1],
                    dst_ref=rsc_ref.at[4 + t],
                    send_sem=rs_send.at[4 + t],
                    recv_sem=rs_recv.at[4 + t],
                    device_id=(left,),
                    device_id_type=pl.DeviceIdType.MESH,
                )
                rdma_l.start()
                rdma_l.wait()
            rdma_r.wait()
        out_ref[...] = (
            rsc_ref[3].astype(jnp.float32)
            + rsc_ref[6].astype(jnp.float32)
            + part_ref[me].astype(jnp.float32)
        )

    return pl.pallas_call(
        body,
        out_shape=jax.ShapeDtypeStruct((B, S_LOC, D), jnp.float32),
        in_specs=[pl.BlockSpec(memory_space=pltpu.VMEM)] * 5,
        out_specs=pl.BlockSpec(memory_space=pltpu.VMEM),
        scratch_shapes=[
            pltpu.VMEM((N_DEV, B, S_LOC, D), jnp.bfloat16),
            pltpu.VMEM((N_DEV, B, S_LOC, D), jnp.bfloat16),
            pltpu.VMEM((2, B, S_LOC, D), jnp.bfloat16),
            pltpu.VMEM((N_DEV - 1, B, S_LOC, D), jnp.bfloat16),
            pltpu.SemaphoreType.DMA((N_DEV - 1,)),
            pltpu.SemaphoreType.DMA((N_DEV - 1,)),
            pltpu.SemaphoreType.DMA((N_DEV - 1,)),
            pltpu.SemaphoreType.DMA((N_DEV - 1,)),
        ],
        compiler_params=pltpu.CompilerParams(collective_id=0),
    )(x, Wq, Wk, Wv, Wo)


# device time: 74115 ns/iter; 1.9144x vs baseline; 1.0593x over previous
import jax
import jax.numpy as jnp
from jax import lax
from jax.experimental import pallas as pl
from jax.experimental.pallas import tpu as pltpu

N_DEV = 8
B = 2
S_LOC = 128
S_GLB = 1024
D = 512
HQ = 4
DH = 64
HD = HQ * DH


def kernel(x, Wq, Wk, Wv, Wo):
    def body(x_ref, wq_ref, wk_ref, wv_ref, wo_ref, out_ref,
             xg_ref, part_ref, sbuf_ref, rsc_ref,
             ag_send, ag_recv, rs_send, rs_recv):
        me = lax.axis_index("i")
        left = (me - 1) % N_DEV
        right = (me + 1) % N_DEV

        xg_ref[me] = x_ref[...].astype(jnp.bfloat16)

        barrier = pltpu.get_barrier_semaphore()
        for nbr in (left, right):
            pl.semaphore_signal(barrier, inc=1, device_id=(nbr,),
                                device_id_type=pl.DeviceIdType.MESH)
        pl.semaphore_wait(barrier, 2)

        for h in range(4):
            c_r = (me - h) % N_DEV
            rdma_r = pltpu.make_async_remote_copy(
                src_ref=xg_ref.at[c_r],
                dst_ref=xg_ref.at[c_r],
                send_sem=ag_send.at[h],
                recv_sem=ag_recv.at[h],
                device_id=(right,),
                device_id_type=pl.DeviceIdType.MESH,
            )
            rdma_r.start()
            if h < 3:
                c_l = (me + h) % N_DEV
                rdma_l = pltpu.make_async_remote_copy(
                    src_ref=xg_ref.at[c_l],
                    dst_ref=xg_ref.at[c_l],
                    send_sem=ag_send.at[4 + h],
                    recv_sem=ag_recv.at[4 + h],
                    device_id=(left,),
                    device_id_type=pl.DeviceIdType.MESH,
                )
                rdma_l.start()
                rdma_l.wait()
            rdma_r.wait()

        lane = lax.broadcasted_iota(jnp.int32, (S_GLB, HD), 1)
        posr = lax.broadcasted_iota(jnp.int32, (S_GLB, HD), 0).astype(jnp.float32)
        j2 = ((lane % DH) // 2).astype(jnp.float32) * 2.0
        inv = jnp.exp(j2 * (-jnp.log(10000.0) / DH))
        ang = posr * inv
        cos = jnp.cos(ang)
        sins = jnp.sin(ang) * jnp.where(lane % 2 == 0, -1.0, 1.0)
        prow = lax.broadcasted_iota(jnp.int32, (HD, HD), 0)
        pcol = lax.broadcasted_iota(jnp.int32, (HD, HD), 1)
        perm = (prow == (pcol + 1 - 2 * (pcol % 2))).astype(jnp.float32)

        bf = jnp.bfloat16
        wq = wq_ref[...].astype(bf)
        wk = wk_ref[...].astype(bf)
        wv = wv_ref[...].astype(bf)
        wo = wo_ref[...].astype(bf)
        perm_bf = perm.astype(bf)
        for b in range(B):
            xb = jnp.concatenate([xg_ref[c, b] for c in range(N_DEV)], axis=0)
            q = jnp.dot(xb, wq, preferred_element_type=jnp.float32)
            k = jnp.dot(xb, wk, preferred_element_type=jnp.float32)
            v = jnp.dot(xb, wv, preferred_element_type=jnp.float32).astype(bf)
            q = (q * cos
                 + jnp.dot(q.astype(bf), perm_bf,
                           preferred_element_type=jnp.float32) * sins).astype(bf)
            k = (k * cos
                 + jnp.dot(k.astype(bf), perm_bf,
                           preferred_element_type=jnp.float32) * sins).astype(bf)
            ob = jnp.zeros((S_GLB, D), jnp.float32)
            for hh in range(HQ):
                sl = slice(hh * DH, (hh + 1) * DH)
                s = lax.dot_general(
                    q[:, sl], k[:, sl], (((1,), (1,)), ((), ())),
                    preferred_element_type=jnp.float32,
                ) * 0.125
                w = jnp.exp(s)
                w = (w / jnp.sum(w, axis=-1, keepdims=True)).astype(bf)
                ctx = jnp.dot(w, v[:, sl],
                              preferred_element_type=jnp.float32).astype(bf)
                ob = ob + jnp.dot(ctx, wo[sl, :],
                                  preferred_element_type=jnp.float32)
            for c in range(N_DEV):
                part_ref[c, b] = ob[c * S_LOC:(c + 1) * S_LOC].astype(jnp.bfloat16)

        for t in range(4):
            if t == 0:
                acc_r = part_ref[(me + 4) % N_DEV]
            else:
                acc_r = (
                    rsc_ref[t - 1].astype(jnp.float32)
                    + part_ref[(me + 4 - t) % N_DEV].astype(jnp.float32)
                ).astype(jnp.bfloat16)
            sbuf_ref[0] = acc_r
            rdma_r = pltpu.make_async_remote_copy(
                src_ref=sbuf_ref.at[0],
                dst_ref=rsc_ref.at[t],
                send_sem=rs_send.at[t],
                recv_sem=rs_recv.at[t],
                device_id=(right,),
                device_id_type=pl.DeviceIdType.MESH,
            )
            rdma_r.start()
            if t < 3:
                if t == 0:
                    acc_l = part_ref[(me - 3) % N_DEV]
                else:
                    acc_l = (
                        rsc_ref[4 + t - 1].astype(jnp.float32)
                        + part_ref[(me - 3 + t) % N_DEV].astype(jnp.float32)
                    ).astype(jnp.bfloat16)
                sbuf_ref[1] = acc_l
                rdma_l = pltpu.make_async_remote_copy(
                    src_ref=sbuf_ref.at[1],
                    dst_ref=rsc_ref.at[4 + t],
                    send_sem=rs_send.at[4 + t],
                    recv_sem=rs_recv.at[4 + t],
                    device_id=(left,),
                    device_id_type=pl.DeviceIdType.MESH,
                )
                rdma_l.start()
                rdma_l.wait()
            rdma_r.wait()
        out_ref[...] = (
            rsc_ref[3].astype(jnp.float32)
            + rsc_ref[6].astype(jnp.float32)
            + part_ref[me].astype(jnp.float32)
        )

    return pl.pallas_call(
        body,
        out_shape=jax.ShapeDtypeStruct((B, S_LOC, D), jnp.float32),
        in_specs=[pl.BlockSpec(memory_space=pltpu.VMEM)] * 5,
        out_specs=pl.BlockSpec(memory_space=pltpu.VMEM),
        scratch_shapes=[
            pltpu.VMEM((N_DEV, B, S_LOC, D), jnp.bfloat16),
            pltpu.VMEM((N_DEV, B, S_LOC, D), jnp.bfloat16),
            pltpu.VMEM((2, B, S_LOC, D), jnp.bfloat16),
            pltpu.VMEM((N_DEV - 1, B, S_LOC, D), jnp.bfloat16),
            pltpu.SemaphoreType.DMA((N_DEV - 1,)),
            pltpu.SemaphoreType.DMA((N_DEV - 1,)),
            pltpu.SemaphoreType.DMA((N_DEV - 1,)),
            pltpu.SemaphoreType.DMA((N_DEV - 1,)),
        ],
        compiler_params=pltpu.CompilerParams(collective_id=0),
    )(x, Wq, Wk, Wv, Wo)


# device time: 68388 ns/iter; 2.0747x vs baseline; 1.0837x over previous
import jax
import jax.numpy as jnp
from jax import lax
from jax.experimental import pallas as pl
from jax.experimental.pallas import tpu as pltpu

N_DEV = 8
B = 2
S_LOC = 128
S_GLB = 1024
D = 512
HQ = 4
DH = 64
HD = HQ * DH

bf = jnp.bfloat16
f32 = jnp.float32


def kernel(x, Wq, Wk, Wv, Wo):
    def body(x_ref, wq_ref, wk_ref, wv_ref, wo_ref, out_ref,
             xg_ref, qs_ref, ks_ref, vs_ref, cos_ref, sin_ref,
             sbr_ref, sbl_ref, rsc_ref,
             ag_send, ag_recv, rs_send, rs_recv):
        me = lax.axis_index("i")
        left = (me - 1) % N_DEV
        right = (me + 1) % N_DEV

        xg_ref[me] = x_ref[...].astype(bf)

        lane = lax.broadcasted_iota(jnp.int32, (S_GLB, HD), 1)
        posr = lax.broadcasted_iota(jnp.int32, (S_GLB, HD), 0).astype(f32)
        j2 = ((lane % DH) // 2).astype(f32) * 2.0
        inv = jnp.exp(j2 * (-jnp.log(10000.0) / DH))
        ang = posr * inv
        cos_ref[...] = jnp.cos(ang)
        sin_ref[...] = jnp.sin(ang) * jnp.where(lane % 2 == 0, -1.0, 1.0)
        prow = lax.broadcasted_iota(jnp.int32, (HD, HD), 0)
        pcol = lax.broadcasted_iota(jnp.int32, (HD, HD), 1)
        perm = (prow == (pcol + 1 - 2 * (pcol % 2))).astype(bf)

        wq = wq_ref[...].astype(bf)
        wk = wk_ref[...].astype(bf)
        wv = wv_ref[...].astype(bf)
        wo = wo_ref[...].astype(bf)

        def qkv_chunk(c):
            r = pl.ds(c * S_LOC, S_LOC)
            cosr = cos_ref[r, :]
            sinr = sin_ref[r, :]
            for b in range(B):
                xcb = xg_ref[c, b]
                q = jnp.dot(xcb, wq, preferred_element_type=f32)
                k = jnp.dot(xcb, wk, preferred_element_type=f32)
                v = jnp.dot(xcb, wv, preferred_element_type=f32)
                q = (q * cosr + jnp.dot(q.astype(bf), perm,
                                        preferred_element_type=f32) * sinr)
                k = (k * cosr + jnp.dot(k.astype(bf), perm,
                                        preferred_element_type=f32) * sinr)
                qs_ref[b, r, :] = (q * 0.125).astype(bf)
                ks_ref[b, r, :] = k.astype(bf)
                vs_ref[b, r, :] = v.astype(bf)

        def attn_chunk(c):
            outs = []
            for b in range(B):
                q_c = qs_ref[b, pl.ds(c * S_LOC, S_LOC), :]
                res = jnp.zeros((S_LOC, D), f32)
                for hh in range(HQ):
                    sl = slice(hh * DH, (hh + 1) * DH)
                    s = lax.dot_general(
                        q_c[:, sl], ks_ref[b, :, sl], (((1,), (1,)), ((), ())),
                        preferred_element_type=f32,
                    )
                    w = jnp.exp(s)
                    w = (w / jnp.sum(w, axis=-1, keepdims=True)).astype(bf)
                    ctx = jnp.dot(w, vs_ref[b, :, sl],
                                  preferred_element_type=f32).astype(bf)
                    res = res + jnp.dot(ctx, wo[sl, :],
                                        preferred_element_type=f32)
                outs.append(res)
            return outs

        barrier = pltpu.get_barrier_semaphore()
        for nbr in (left, right):
            pl.semaphore_signal(barrier, inc=1, device_id=(nbr,),
                                device_id_type=pl.DeviceIdType.MESH)
        pl.semaphore_wait(barrier, 2)

        for h in range(4):
            c_r = (me - h) % N_DEV
            rdma_r = pltpu.make_async_remote_copy(
                src_ref=xg_ref.at[c_r],
                dst_ref=xg_ref.at[c_r],
                send_sem=ag_send.at[h],
                recv_sem=ag_recv.at[h],
                device_id=(right,),
                device_id_type=pl.DeviceIdType.MESH,
            )
            rdma_r.start()
            if h < 3:
                c_l = (me + h) % N_DEV
                rdma_l = pltpu.make_async_remote_copy(
                    src_ref=xg_ref.at[c_l],
                    dst_ref=xg_ref.at[c_l],
                    send_sem=ag_send.at[4 + h],
                    recv_sem=ag_recv.at[4 + h],
                    device_id=(left,),
                    device_id_type=pl.DeviceIdType.MESH,
                )
                rdma_l.start()
            if h == 0:
                qkv_chunk(me)
            else:
                qkv_chunk((me - h) % N_DEV)
                qkv_chunk((me + h) % N_DEV)
            if h < 3:
                rdma_l.wait()
            rdma_r.wait()
        qkv_chunk((me - 4) % N_DEV)

        rs_descs = []
        acc_r = attn_chunk((me + 4) % N_DEV)
        acc_l = attn_chunk((me - 3) % N_DEV)
        nxt_r = None
        for t in range(4):
            for b in range(B):
                sbr_ref[t, b] = acc_r[b].astype(bf)
            rdma_r = pltpu.make_async_remote_copy(
                src_ref=sbr_ref.at[t],
                dst_ref=rsc_ref.at[t],
                send_sem=rs_send.at[t],
                recv_sem=rs_recv.at[t],
                device_id=(right,),
                device_id_type=pl.DeviceIdType.MESH,
            )
            rdma_r.start()
            rs_descs.append(rdma_r)
            if t < 3:
                for b in range(B):
                    sbl_ref[t, b] = acc_l[b].astype(bf)
                rdma_l = pltpu.make_async_remote_copy(
                    src_ref=sbl_ref.at[t],
                    dst_ref=rsc_ref.at[4 + t],
                    send_sem=rs_send.at[4 + t],
                    recv_sem=rs_recv.at[4 + t],
                    device_id=(left,),
                    device_id_type=pl.DeviceIdType.MESH,
                )
                rdma_l.start()
                rs_descs.append(rdma_l)
            nxt_r = attn_chunk((me + 3 - t) % N_DEV)
            if t < 2:
                nxt_l = attn_chunk((me - 2 + t) % N_DEV)
            if t < 3:
                rdma_l.wait_recv()
            rdma_r.wait_recv()
            if t < 3:
                acc_r = [rsc_ref[t, b].astype(f32) + nxt_r[b] for b in range(B)]
            if t < 2:
                acc_l = [rsc_ref[4 + t, b].astype(f32) + nxt_l[b]
                         for b in range(B)]

        for b in range(B):
            out_ref[b] = (rsc_ref[3, b].astype(f32)
                          + rsc_ref[6, b].astype(f32)
                          + nxt_r[b])
        for d in rs_descs:
            d.wait_send()

    return pl.pallas_call(
        body,
        out_shape=jax.ShapeDtypeStruct((B, S_LOC, D), f32),
        in_specs=[pl.BlockSpec(memory_space=pltpu.VMEM)] * 5,
        out_specs=pl.BlockSpec(memory_space=pltpu.VMEM),
        scratch_shapes=[
            pltpu.VMEM((N_DEV, B, S_LOC, D), bf),
            pltpu.VMEM((B, S_GLB, HD), bf),
            pltpu.VMEM((B, S_GLB, HD), bf),
            pltpu.VMEM((B, S_GLB, HD), bf),
            pltpu.VMEM((S_GLB, HD), f32),
            pltpu.VMEM((S_GLB, HD), f32),
            pltpu.VMEM((4, B, S_LOC, D), bf),
            pltpu.VMEM((3, B, S_LOC, D), bf),
            pltpu.VMEM((N_DEV - 1, B, S_LOC, D), bf),
            pltpu.SemaphoreType.DMA((N_DEV - 1,)),
            pltpu.SemaphoreType.DMA((N_DEV - 1,)),
            pltpu.SemaphoreType.DMA((N_DEV - 1,)),
            pltpu.SemaphoreType.DMA((N_DEV - 1,)),
        ],
        compiler_params=pltpu.CompilerParams(collective_id=0),
    )(x, Wq, Wk, Wv, Wo)


# device time: 63368 ns/iter; 2.2390x vs baseline; 1.0792x over previous
import jax
import jax.numpy as jnp
from jax import lax
from jax.experimental import pallas as pl
from jax.experimental.pallas import tpu as pltpu

N_DEV = 8
B = 2
S_LOC = 128
S_GLB = 1024
D = 512
HQ = 4
DH = 64
HD = HQ * DH

bf = jnp.bfloat16
f32 = jnp.float32


def kernel(x, Wq, Wk, Wv, Wo):
    def body(x_ref, wq_ref, wk_ref, wv_ref, wo_ref, out_ref,
             xg_ref, qs_ref, ks_ref, vs_ref, cos_ref, sin_ref,
             sbr_ref, sbl_ref, rsc_ref,
             ag_send, ag_recv, rs_send, rs_recv):
        me = lax.axis_index("i")
        left = (me - 1) % N_DEV
        right = (me + 1) % N_DEV

        xg_ref[me] = x_ref[...].astype(bf)

        lane = lax.broadcasted_iota(jnp.int32, (S_GLB, HD), 1)
        posr = lax.broadcasted_iota(jnp.int32, (S_GLB, HD), 0).astype(f32)
        j2 = ((lane % DH) // 2).astype(f32) * 2.0
        inv = jnp.exp(j2 * (-jnp.log(10000.0) / DH))
        ang = posr * inv
        cos_ref[...] = jnp.cos(ang)
        sin_ref[...] = jnp.sin(ang) * jnp.where(lane % 2 == 0, -1.0, 1.0)
        prow = lax.broadcasted_iota(jnp.int32, (HD, HD), 0)
        pcol = lax.broadcasted_iota(jnp.int32, (HD, HD), 1)
        perm = (prow == (pcol + 1 - 2 * (pcol % 2))).astype(bf)

        wq = wq_ref[...].astype(bf)
        wk = wk_ref[...].astype(bf)
        wv = wv_ref[...].astype(bf)
        wo = wo_ref[...].astype(bf)

        def qkv_chunk(c):
            r = pl.ds(c * S_LOC, S_LOC)
            cosr = cos_ref[r, :]
            sinr = sin_ref[r, :]
            for b in range(B):
                xcb = xg_ref[c, b]
                q = jnp.dot(xcb, wq, preferred_element_type=f32)
                k = jnp.dot(xcb, wk, preferred_element_type=f32)
                v = jnp.dot(xcb, wv, preferred_element_type=f32)
                q = (q * cosr + jnp.dot(q.astype(bf), perm,
                                        preferred_element_type=f32) * sinr)
                k = (k * cosr + jnp.dot(k.astype(bf), perm,
                                        preferred_element_type=f32) * sinr)
                qs_ref[b, r, :] = (q * (0.125 * 1.4426950408889634)).astype(bf)
                ks_ref[b, r, :] = k.astype(bf)
                vs_ref[b, r, :] = v.astype(bf)

        def attn_chunk(c):
            outs = []
            for b in range(B):
                q_c = qs_ref[b, pl.ds(c * S_LOC, S_LOC), :]
                res = jnp.zeros((S_LOC, D), f32)
                for hh in range(HQ):
                    sl = slice(hh * DH, (hh + 1) * DH)
                    s = lax.dot_general(
                        q_c[:, sl], ks_ref[b, :, sl], (((1,), (1,)), ((), ())),
                        preferred_element_type=f32,
                    )
                    w = jnp.exp2(s)
                    dinv = 1.0 / jnp.sum(w, axis=-1, keepdims=True)
                    ctx = jnp.dot(w.astype(bf), vs_ref[b, :, sl],
                                  preferred_element_type=f32)
                    res = res + jnp.dot((ctx * dinv).astype(bf), wo[sl, :],
                                        preferred_element_type=f32)
                outs.append(res)
            return outs

        barrier = pltpu.get_barrier_semaphore()
        for nbr in (left, right):
            pl.semaphore_signal(barrier, inc=1, device_id=(nbr,),
                                device_id_type=pl.DeviceIdType.MESH)
        pl.semaphore_wait(barrier, 2)

        for h in range(4):
            c_r = (me - h) % N_DEV
            rdma_r = pltpu.make_async_remote_copy(
                src_ref=xg_ref.at[c_r],
                dst_ref=xg_ref.at[c_r],
                send_sem=ag_send.at[h],
                recv_sem=ag_recv.at[h],
                device_id=(right,),
                device_id_type=pl.DeviceIdType.MESH,
            )
            rdma_r.start()
            if h < 3:
                c_l = (me + h) % N_DEV
                rdma_l = pltpu.make_async_remote_copy(
                    src_ref=xg_ref.at[c_l],
                    dst_ref=xg_ref.at[c_l],
                    send_sem=ag_send.at[4 + h],
                    recv_sem=ag_recv.at[4 + h],
                    device_id=(left,),
                    device_id_type=pl.DeviceIdType.MESH,
                )
                rdma_l.start()
            if h == 0:
                qkv_chunk(me)
            else:
                qkv_chunk((me - h) % N_DEV)
                qkv_chunk((me + h) % N_DEV)
            if h < 3:
                rdma_l.wait()
            rdma_r.wait()
        qkv_chunk((me - 4) % N_DEV)

        rs_descs = []
        acc_r = attn_chunk((me + 4) % N_DEV)
        acc_l = attn_chunk((me - 3) % N_DEV)
        nxt_r = None
        for t in range(4):
            for b in range(B):
                sbr_ref[t, b] = acc_r[b].astype(bf)
            rdma_r = pltpu.make_async_remote_copy(
                src_ref=sbr_ref.at[t],
                dst_ref=rsc_ref.at[t],
                send_sem=rs_send.at[t],
                recv_sem=rs_recv.at[t],
                device_id=(right,),
                device_id_type=pl.DeviceIdType.MESH,
            )
            rdma_r.start()
            rs_descs.append(rdma_r)
            if t < 3:
                for b in range(B):
                    sbl_ref[t, b] = acc_l[b].astype(bf)
                rdma_l = pltpu.make_async_remote_copy(
                    src_ref=sbl_ref.at[t],
                    dst_ref=rsc_ref.at[4 + t],
                    send_sem=rs_send.at[4 + t],
                    recv_sem=rs_recv.at[4 + t],
                    device_id=(left,),
                    device_id_type=pl.DeviceIdType.MESH,
                )
                rdma_l.start()
                rs_descs.append(rdma_l)
            nxt_r = attn_chunk((me + 3 - t) % N_DEV)
            if t < 2:
                nxt_l = attn_chunk((me - 2 + t) % N_DEV)
            if t < 3:
                rdma_l.wait_recv()
            rdma_r.wait_recv()
            if t < 3:
                acc_r = [rsc_ref[t, b].astype(f32) + nxt_r[b] for b in range(B)]
            if t < 2:
                acc_l = [rsc_ref[4 + t, b].astype(f32) + nxt_l[b]
                         for b in range(B)]

        for b in range(B):
            out_ref[b] = (rsc_ref[3, b].astype(f32)
                          + rsc_ref[6, b].astype(f32)
                          + nxt_r[b])
        for d in rs_descs:
            d.wait_send()

    return pl.pallas_call(
        body,
        out_shape=jax.ShapeDtypeStruct((B, S_LOC, D), f32),
        in_specs=[pl.BlockSpec(memory_space=pltpu.VMEM)] * 5,
        out_specs=pl.BlockSpec(memory_space=pltpu.VMEM),
        scratch_shapes=[
            pltpu.VMEM((N_DEV, B, S_LOC, D), bf),
            pltpu.VMEM((B, S_GLB, HD), bf),
            pltpu.VMEM((B, S_GLB, HD), bf),
            pltpu.VMEM((B, S_GLB, HD), bf),
            pltpu.VMEM((S_GLB, HD), f32),
            pltpu.VMEM((S_GLB, HD), f32),
            pltpu.VMEM((4, B, S_LOC, D), bf),
            pltpu.VMEM((3, B, S_LOC, D), bf),
            pltpu.VMEM((N_DEV - 1, B, S_LOC, D), bf),
            pltpu.SemaphoreType.DMA((N_DEV - 1,)),
            pltpu.SemaphoreType.DMA((N_DEV - 1,)),
            pltpu.SemaphoreType.DMA((N_DEV - 1,)),
            pltpu.SemaphoreType.DMA((N_DEV - 1,)),
        ],
        compiler_params=pltpu.CompilerParams(collective_id=0),
    )(x, Wq, Wk, Wv, Wo)


# device time: 63128 ns/iter; 2.2476x vs baseline; 1.0038x over previous
import jax
import jax.numpy as jnp
from jax import lax
from jax.experimental import pallas as pl
from jax.experimental.pallas import tpu as pltpu

N_DEV = 8
B = 2
S_LOC = 128
S_GLB = 1024
D = 512
HQ = 4
DH = 64
HD = HQ * DH

bf = jnp.bfloat16
f32 = jnp.float32


def kernel(x, Wq, Wk, Wv, Wo):
    def body(x_ref, wq_ref, wk_ref, wv_ref, wo_ref, out_ref,
             xg_ref, qs_ref, ks_ref, vs_ref, cos_ref, sin_ref,
             sbr_ref, sbl_ref, rsc_ref,
             ag_send, ag_recv, rs_send, rs_recv):
        me = lax.axis_index("i")
        left = (me - 1) % N_DEV
        right = (me + 1) % N_DEV

        xg_ref[me] = x_ref[...].astype(bf)

        lane = lax.broadcasted_iota(jnp.int32, (S_GLB, HD), 1)
        posr = lax.broadcasted_iota(jnp.int32, (S_GLB, HD), 0).astype(f32)
        j2 = ((lane % DH) // 2).astype(f32) * 2.0
        inv = jnp.exp(j2 * (-jnp.log(10000.0) / DH))
        ang = posr * inv
        cos_ref[...] = jnp.cos(ang)
        sin_ref[...] = jnp.sin(ang) * jnp.where(lane % 2 == 0, -1.0, 1.0)
        prow = lax.broadcasted_iota(jnp.int32, (HD, HD), 0)
        pcol = lax.broadcasted_iota(jnp.int32, (HD, HD), 1)
        perm = (prow == (pcol + 1 - 2 * (pcol % 2))).astype(bf)

        wq = wq_ref[...].astype(bf)
        wk = wk_ref[...].astype(bf)
        wv = wv_ref[...].astype(bf)
        wo = wo_ref[...].astype(bf)

        def qkv_chunk(c):
            r = pl.ds(c * S_LOC, S_LOC)
            cosr = cos_ref[r, :]
            sinr = sin_ref[r, :]
            for b in range(B):
                xcb = xg_ref[c, b]
                q = jnp.dot(xcb, wq, preferred_element_type=f32)
                k = jnp.dot(xcb, wk, preferred_element_type=f32)
                v = jnp.dot(xcb, wv, preferred_element_type=f32)
                q = (q * cosr + jnp.dot(q.astype(bf), perm,
                                        preferred_element_type=f32) * sinr)
                k = (k * cosr + jnp.dot(k.astype(bf), perm,
                                        preferred_element_type=f32) * sinr)
                qs_ref[b, r, :] = (q * (0.125 * 1.4426950408889634)).astype(bf)
                ks_ref[b, r, :] = k.astype(bf)
                vs_ref[b, r, :] = v.astype(bf)

        def attn_chunk(c):
            outs = []
            for b in range(B):
                q_c = qs_ref[b, pl.ds(c * S_LOC, S_LOC), :]
                res = jnp.zeros((S_LOC, D), f32)
                for hh in range(HQ):
                    sl = slice(hh * DH, (hh + 1) * DH)
                    s = lax.dot_general(
                        q_c[:, sl], ks_ref[b, :, sl], (((1,), (1,)), ((), ())),
                        preferred_element_type=f32,
                    )
                    w = jnp.exp2(s)
                    dinv = 1.0 / jnp.sum(w, axis=-1, keepdims=True)
                    ctx = jnp.dot(w.astype(bf), vs_ref[b, :, sl],
                                  preferred_element_type=f32)
                    res = res + jnp.dot((ctx * dinv).astype(bf), wo[sl, :],
                                        preferred_element_type=f32)
                outs.append(res)
            return outs

        barrier = pltpu.get_barrier_semaphore()
        for nbr in (left, right):
            pl.semaphore_signal(barrier, inc=1, device_id=(nbr,),
                                device_id_type=pl.DeviceIdType.MESH)
        pl.semaphore_wait(barrier, 2)

        ag_descs = []

        def ag_start(c, sem_i, dev):
            d = pltpu.make_async_remote_copy(
                src_ref=xg_ref.at[c],
                dst_ref=xg_ref.at[c],
                send_sem=ag_send.at[sem_i],
                recv_sem=ag_recv.at[sem_i],
                device_id=(dev,),
                device_id_type=pl.DeviceIdType.MESH,
            )
            d.start()
            ag_descs.append(d)
            return d

        rd = [None] * 4
        ld = [None] * 3
        rd[0] = ag_start(me, 0, right)
        ld[0] = ag_start(me, 4, left)
        qkv_chunk(me)
        for h in range(1, 4):
            rd[h - 1].wait_recv()
            ld[h - 1].wait_recv()
            rd[h] = ag_start((me - h) % N_DEV, h, right)
            if h < 3:
                ld[h] = ag_start((me + h) % N_DEV, 4 + h, left)
            qkv_chunk((me - h) % N_DEV)
            qkv_chunk((me + h) % N_DEV)
        rd[3].wait_recv()
        qkv_chunk((me - 4) % N_DEV)
        for d in ag_descs:
            d.wait_send()

        rs_descs = []
        acc_r = attn_chunk((me + 4) % N_DEV)
        acc_l = attn_chunk((me - 3) % N_DEV)
        nxt_r = None
        for t in range(4):
            for b in range(B):
                sbr_ref[t, b] = acc_r[b].astype(bf)
            rdma_r = pltpu.make_async_remote_copy(
                src_ref=sbr_ref.at[t],
                dst_ref=rsc_ref.at[t],
                send_sem=rs_send.at[t],
                recv_sem=rs_recv.at[t],
                device_id=(right,),
                device_id_type=pl.DeviceIdType.MESH,
            )
            rdma_r.start()
            rs_descs.append(rdma_r)
            if t < 3:
                for b in range(B):
                    sbl_ref[t, b] = acc_l[b].astype(bf)
                rdma_l = pltpu.make_async_remote_copy(
                    src_ref=sbl_ref.at[t],
                    dst_ref=rsc_ref.at[4 + t],
                    send_sem=rs_send.at[4 + t],
                    recv_sem=rs_recv.at[4 + t],
                    device_id=(left,),
                    device_id_type=pl.DeviceIdType.MESH,
                )
                rdma_l.start()
                rs_descs.append(rdma_l)
            nxt_r = attn_chunk((me + 3 - t) % N_DEV)
            if t < 2:
                nxt_l = attn_chunk((me - 2 + t) % N_DEV)
            if t < 3:
                rdma_l.wait_recv()
            rdma_r.wait_recv()
            if t < 3:
                acc_r = [rsc_ref[t, b].astype(f32) + nxt_r[b] for b in range(B)]
            if t < 2:
                acc_l = [rsc_ref[4 + t, b].astype(f32) + nxt_l[b]
                         for b in range(B)]

        for b in range(B):
            out_ref[b] = (rsc_ref[3, b].astype(f32)
                          + rsc_ref[6, b].astype(f32)
                          + nxt_r[b])
        for d in rs_descs:
            d.wait_send()

    return pl.pallas_call(
        body,
        out_shape=jax.ShapeDtypeStruct((B, S_LOC, D), f32),
        in_specs=[pl.BlockSpec(memory_space=pltpu.VMEM)] * 5,
        out_specs=pl.BlockSpec(memory_space=pltpu.VMEM),
        scratch_shapes=[
            pltpu.VMEM((N_DEV, B, S_LOC, D), bf),
            pltpu.VMEM((B, S_GLB, HD), bf),
            pltpu.VMEM((B, S_GLB, HD), bf),
            pltpu.VMEM((B, S_GLB, HD), bf),
            pltpu.VMEM((S_GLB, HD), f32),
            pltpu.VMEM((S_GLB, HD), f32),
            pltpu.VMEM((4, B, S_LOC, D), bf),
            pltpu.VMEM((3, B, S_LOC, D), bf),
            pltpu.VMEM((N_DEV - 1, B, S_LOC, D), bf),
            pltpu.SemaphoreType.DMA((N_DEV - 1,)),
            pltpu.SemaphoreType.DMA((N_DEV - 1,)),
            pltpu.SemaphoreType.DMA((N_DEV - 1,)),
            pltpu.SemaphoreType.DMA((N_DEV - 1,)),
        ],
        compiler_params=pltpu.CompilerParams(collective_id=0),
    )(x, Wq, Wk, Wv, Wo)


# device time: 60419 ns/iter; 2.3483x vs baseline; 1.0448x over previous
import jax
import jax.numpy as jnp
from jax import lax
from jax.experimental import pallas as pl
from jax.experimental.pallas import tpu as pltpu

N_DEV = 8
B = 2
S_LOC = 128
S_GLB = 1024
D = 512
HQ = 4
DH = 64
HD = HQ * DH

bf = jnp.bfloat16
f32 = jnp.float32


def kernel(x, Wq, Wk, Wv, Wo):
    def body(x_ref, wq_ref, wk_ref, wv_ref, wo_ref, out_ref,
             xg_ref, qs_ref, ks_ref, vs_ref, cos_ref, sin_ref,
             sbr_ref, sbl_ref, rsc_ref,
             ag_send, ag_recv, rs_send, rs_recv):
        me = lax.axis_index("i")
        left = (me - 1) % N_DEV
        right = (me + 1) % N_DEV

        xg_ref[me] = x_ref[...].astype(bf)

        lane = lax.broadcasted_iota(jnp.int32, (S_GLB, HD), 1)
        posr = lax.broadcasted_iota(jnp.int32, (S_GLB, HD), 0).astype(f32)
        j2 = ((lane % DH) // 2).astype(f32) * 2.0
        inv = jnp.exp(j2 * (-jnp.log(10000.0) / DH))
        ang = posr * inv
        cos_ref[...] = jnp.cos(ang)
        sin_ref[...] = jnp.sin(ang) * jnp.where(lane % 2 == 0, -1.0, 1.0)
        prow = lax.broadcasted_iota(jnp.int32, (HD, HD), 0)
        pcol = lax.broadcasted_iota(jnp.int32, (HD, HD), 1)
        perm = (prow == (pcol + 1 - 2 * (pcol % 2))).astype(bf)

        wq = wq_ref[...].astype(bf)
        wk = wk_ref[...].astype(bf)
        wv = wv_ref[...].astype(bf)
        wo = wo_ref[...].astype(bf)

        def qkv_chunk(c):
            r = pl.ds(c * S_LOC, S_LOC)
            cosr = cos_ref[r, :]
            sinr = sin_ref[r, :]
            cos2 = jnp.concatenate([cosr, cosr], axis=0)
            sin2 = jnp.concatenate([sinr, sinr], axis=0)
            xc = xg_ref[c].reshape(B * S_LOC, D)
            q = jnp.dot(xc, wq, preferred_element_type=f32)
            k = jnp.dot(xc, wk, preferred_element_type=f32)
            v = jnp.dot(xc, wv, preferred_element_type=f32)
            q = (q * cos2 + jnp.dot(q.astype(bf), perm,
                                    preferred_element_type=f32) * sin2)
            k = (k * cos2 + jnp.dot(k.astype(bf), perm,
                                    preferred_element_type=f32) * sin2)
            qb = (q * (0.125 * 1.4426950408889634)).astype(bf)
            kb = k.astype(bf)
            vb = v.astype(bf)
            for b in range(B):
                rows = slice(b * S_LOC, (b + 1) * S_LOC)
                qs_ref[b, r, :] = qb[rows]
                ks_ref[b, r, :] = kb[rows]
                vs_ref[b, r, :] = vb[rows]

        def attn_chunk(c):
            outs = []
            for b in range(B):
                q_c = qs_ref[b, pl.ds(c * S_LOC, S_LOC), :]
                ctxs = []
                for hh in range(HQ):
                    sl = slice(hh * DH, (hh + 1) * DH)
                    s = lax.dot_general(
                        q_c[:, sl], ks_ref[b, :, sl], (((1,), (1,)), ((), ())),
                        preferred_element_type=f32,
                    )
                    w = jnp.exp2(s)
                    dinv = 1.0 / jnp.sum(w, axis=-1, keepdims=True)
                    ctx = jnp.dot(w.astype(bf), vs_ref[b, :, sl],
                                  preferred_element_type=f32)
                    ctxs.append((ctx * dinv).astype(bf))
                ctx_all = jnp.concatenate(ctxs, axis=1)
                outs.append(jnp.dot(ctx_all, wo, preferred_element_type=f32))
            return outs

        barrier = pltpu.get_barrier_semaphore()
        for nbr in (left, right):
            pl.semaphore_signal(barrier, inc=1, device_id=(nbr,),
                                device_id_type=pl.DeviceIdType.MESH)
        pl.semaphore_wait(barrier, 2)

        ag_descs = []

        def ag_start(c, sem_i, dev):
            d = pltpu.make_async_remote_copy(
                src_ref=xg_ref.at[c],
                dst_ref=xg_ref.at[c],
                send_sem=ag_send.at[sem_i],
                recv_sem=ag_recv.at[sem_i],
                device_id=(dev,),
                device_id_type=pl.DeviceIdType.MESH,
            )
            d.start()
            ag_descs.append(d)
            return d

        rd = [None] * 4
        ld = [None] * 3
        rd[0] = ag_start(me, 0, right)
        ld[0] = ag_start(me, 4, left)
        qkv_chunk(me)
        for h in range(1, 4):
            rd[h - 1].wait_recv()
            ld[h - 1].wait_recv()
            rd[h] = ag_start((me - h) % N_DEV, h, right)
            if h < 3:
                ld[h] = ag_start((me + h) % N_DEV, 4 + h, left)
            qkv_chunk((me - h) % N_DEV)
            qkv_chunk((me + h) % N_DEV)
        rd[3].wait_recv()
        qkv_chunk((me - 4) % N_DEV)
        for d in ag_descs:
            d.wait_send()

        rs_descs = []
        acc_r = attn_chunk((me + 4) % N_DEV)
        acc_l = attn_chunk((me - 3) % N_DEV)
        nxt_r = None
        for t in range(4):
            for b in range(B):
                sbr_ref[t, b] = acc_r[b].astype(bf)
            rdma_r = pltpu.make_async_remote_copy(
                src_ref=sbr_ref.at[t],
                dst_ref=rsc_ref.at[t],
                send_sem=rs_send.at[t],
                recv_sem=rs_recv.at[t],
                device_id=(right,),
                device_id_type=pl.DeviceIdType.MESH,
            )
            rdma_r.start()
            rs_descs.append(rdma_r)
            if t < 3:
                for b in range(B):
                    sbl_ref[t, b] = acc_l[b].astype(bf)
                rdma_l = pltpu.make_async_remote_copy(
                    src_ref=sbl_ref.at[t],
                    dst_ref=rsc_ref.at[4 + t],
                    send_sem=rs_send.at[4 + t],
                    recv_sem=rs_recv.at[4 + t],
                    device_id=(left,),
                    device_id_type=pl.DeviceIdType.MESH,
                )
                rdma_l.start()
                rs_descs.append(rdma_l)
            nxt_r = attn_chunk((me + 3 - t) % N_DEV)
            if t < 2:
                nxt_l = attn_chunk((me - 2 + t) % N_DEV)
            if t < 3:
                rdma_l.wait_recv()
            rdma_r.wait_recv()
            if t < 3:
                acc_r = [rsc_ref[t, b].astype(f32) + nxt_r[b] for b in range(B)]
            if t < 2:
                acc_l = [rsc_ref[4 + t, b].astype(f32) + nxt_l[b]
                         for b in range(B)]

        for b in range(B):
            out_ref[b] = (rsc_ref[3, b].astype(f32)
                          + rsc_ref[6, b].astype(f32)
                          + nxt_r[b])
        for d in rs_descs:
            d.wait_send()

    return pl.pallas_call(
        body,
        out_shape=jax.ShapeDtypeStruct((B, S_LOC, D), f32),
        in_specs=[pl.BlockSpec(memory_space=pltpu.VMEM)] * 5,
        out_specs=pl.BlockSpec(memory_space=pltpu.VMEM),
        scratch_shapes=[
            pltpu.VMEM((N_DEV, B, S_LOC, D), bf),
            pltpu.VMEM((B, S_GLB, HD), bf),
            pltpu.VMEM((B, S_GLB, HD), bf),
            pltpu.VMEM((B, S_GLB, HD), bf),
            pltpu.VMEM((S_GLB, HD), f32),
            pltpu.VMEM((S_GLB, HD), f32),
            pltpu.VMEM((4, B, S_LOC, D), bf),
            pltpu.VMEM((3, B, S_LOC, D), bf),
            pltpu.VMEM((N_DEV - 1, B, S_LOC, D), bf),
            pltpu.SemaphoreType.DMA((N_DEV - 1,)),
            pltpu.SemaphoreType.DMA((N_DEV - 1,)),
            pltpu.SemaphoreType.DMA((N_DEV - 1,)),
            pltpu.SemaphoreType.DMA((N_DEV - 1,)),
        ],
        compiler_params=pltpu.CompilerParams(collective_id=0),
    )(x, Wq, Wk, Wv, Wo)
